# baseline (speedup 1.0000x reference)
import functools

import jax
import jax.numpy as jnp
from jax import lax
from jax.experimental import pallas as pl
from jax.experimental.pallas import tpu as pltpu
from jax.experimental.pallas import tpu_sc as plsc

_NUM_CORES = 2
_NUM_SUBCORES = 16
_NUM_WORKERS = _NUM_CORES * _NUM_SUBCORES
_CHUNK = 800
_GRP = 2
_SUPER = _CHUNK * _GRP
_PAD_D = 8
_FEAT = 6
_LANES = 16


def _sc_gather_compact(table, idx2d, n_tokens):
    per_w = n_tokens // _NUM_WORKERS
    nsteps = per_w // _CHUNK
    nsuper = per_w // _SUPER
    blocks = _SUPER // _PAD_D
    mesh = plsc.VectorSubcoreMesh(core_axis_name="c", subcore_axis_name="s")

    @functools.partial(
        pl.kernel,
        mesh=mesh,
        out_type=jax.ShapeDtypeStruct((n_tokens * _FEAT,), jnp.float32),
        scratch_types=[
            pltpu.VMEM((nsteps, _CHUNK), jnp.int32),
            pltpu.VMEM((2, _SUPER, _PAD_D), jnp.float32),
            pltpu.VMEM((2, _SUPER * _FEAT), jnp.float32),
            pltpu.SemaphoreType.DMA,
            pltpu.SemaphoreType.DMA,
        ],
        compiler_params=pltpu.CompilerParams(
            use_tc_tiling_on_sc=False, needs_layout_passes=False),
    )
    def k(table_hbm, idx_hbm, out_hbm, idx_v, rows_v, cbuf_v, gsem, osem):
        wid = lax.axis_index("s") * _NUM_CORES + lax.axis_index("c")
        base = wid * per_w
        pltpu.sync_copy(idx_hbm.at[pl.ds(wid * nsteps, nsteps)], idx_v)

        lane = lax.iota(jnp.int32, _LANES)
        def _divmod6(j):
            q = lax.shift_right_logical(j * 43691, 18)
            return q, j - q * _FEAT
        r_pat, c_pat = [], []
        for ph in range(3):
            q, r = _divmod6(lane + ph * _LANES)
            r_pat.append(q)
            c_pat.append(r)

        def fire(sp):
            slot = sp % 2
            return [
                pltpu.async_copy(
                    table_hbm.at[idx_v.at[sp * _GRP + g]],
                    rows_v.at[slot].at[pl.ds(g * _CHUNK, _CHUNK)],
                    gsem,
                )
                for g in range(_GRP)
            ]

        def compact(slot):
            rows = rows_v.at[slot]
            cb = cbuf_v.at[slot]

            def body(i2, carry):
                rbase = i2 * _PAD_D
                obase = i2 * (3 * _LANES)
                for ph in range(3):
                    v = plsc.load_gather(rows, [r_pat[ph] + rbase, c_pat[ph]])
                    cb[pl.ds(obase + ph * _LANES, _LANES)] = v
                return carry

            lax.fori_loop(0, blocks, body, 0)

        out_copies = [None, None]
        gathers = fire(0)
        for sp in range(nsuper):
            slot = sp % 2
            if sp + 1 < nsuper:
                gathers_next = fire(sp + 1)
            for g in gathers:
                g.wait()
            if out_copies[slot] is not None:
                out_copies[slot].wait()
            compact(slot)
            out_copies[slot] = pltpu.async_copy(
                cbuf_v.at[slot],
                out_hbm.at[pl.ds((base + sp * _SUPER) * _FEAT, _SUPER * _FEAT)],
                osem,
            )
            if sp + 1 < nsuper:
                gathers = gathers_next
        for oc in out_copies:
            if oc is not None:
                oc.wait()

    return k(table, idx2d)


def kernel(token_ids, frequencies, amplitudes):
    b, s = token_ids.shape
    v, nw = frequencies.shape
    n = b * s
    feat = 2 * nw
    table = jnp.concatenate(
        [frequencies, amplitudes,
         jnp.zeros((v, _PAD_D - feat), jnp.float32)], axis=1)
    idx_flat = token_ids.reshape(-1).astype(jnp.int32)
    idx_flat = lax.optimization_barrier(idx_flat)
    idx2d = idx_flat.reshape(n // _CHUNK, _CHUNK)
    out = _sc_gather_compact(table, idx2d, n)
    return out.reshape(b, s, feat)


# speedup vs baseline: 1.3469x; 1.3469x over previous
import functools

import jax
import jax.numpy as jnp
from jax import lax
from jax.experimental import pallas as pl
from jax.experimental.pallas import tpu as pltpu
from jax.experimental.pallas import tpu_sc as plsc

_NUM_CORES = 2
_NUM_SUBCORES = 16
_NUM_WORKERS = _NUM_CORES * _NUM_SUBCORES
_CHUNK = 800
_GRP = 2
_SUPER = _CHUNK * _GRP
_PAD_D = 8
_FEAT = 6
_LANES = 16


def _sc_gather_compact(table, idx2d, n_tokens):
    per_w = n_tokens // _NUM_WORKERS
    nsteps = per_w // _CHUNK
    nsuper = per_w // _SUPER
    blocks = _SUPER // _PAD_D
    mesh = plsc.VectorSubcoreMesh(core_axis_name="c", subcore_axis_name="s")

    @functools.partial(
        pl.kernel,
        mesh=mesh,
        out_type=jax.ShapeDtypeStruct((n_tokens * _FEAT,), jnp.float32),
        scratch_types=[
            pltpu.VMEM((nsteps, _CHUNK), jnp.int32),
            pltpu.VMEM((2, _SUPER, _PAD_D), jnp.float32),
            pltpu.VMEM((2, _SUPER * _FEAT), jnp.float32),
            pltpu.SemaphoreType.DMA,
            pltpu.SemaphoreType.DMA,
        ],
        compiler_params=pltpu.CompilerParams(
            use_tc_tiling_on_sc=False, needs_layout_passes=False),
    )
    def k(table_hbm, idx_hbm, out_hbm, idx_v, rows_v, cbuf_v, gsem, osem):
        wid = lax.axis_index("s") * _NUM_CORES + lax.axis_index("c")
        base = wid * per_w
        pltpu.sync_copy(idx_hbm.at[pl.ds(wid * nsteps, nsteps)], idx_v)

        lane = lax.iota(jnp.int32, _LANES)
        def _divmod6(j):
            q = lax.shift_right_logical(j * 43691, 18)
            return q, j - q * _FEAT
        r_pat, c_pat = [], []
        for ph in range(3):
            q, r = _divmod6(lane + ph * _LANES)
            r_pat.append(q)
            c_pat.append(r)

        def fire(sp):
            slot = sp % 2
            return [
                pltpu.async_copy(
                    table_hbm.at[idx_v.at[sp * _GRP + g]],
                    rows_v.at[slot].at[pl.ds(g * _CHUNK, _CHUNK)],
                    gsem,
                )
                for g in range(_GRP)
            ]

        def compact(slot):
            rows = rows_v.at[slot]
            cb = cbuf_v.at[slot]

            def body(i2, carry):
                rbase = i2 * _PAD_D
                obase = i2 * (3 * _LANES)
                for ph in range(3):
                    v = plsc.load_gather(rows, [r_pat[ph] + rbase, c_pat[ph]])
                    cb[pl.ds(obase + ph * _LANES, _LANES)] = v
                return carry

            lax.fori_loop(0, blocks, body, 0)

        out_copies = [None, None]
        gathers = fire(0)
        for sp in range(nsuper):
            slot = sp % 2
            if sp + 1 < nsuper:
                gathers_next = fire(sp + 1)
            for g in gathers:
                g.wait()
            if out_copies[slot] is not None:
                out_copies[slot].wait()
            compact(slot)
            out_copies[slot] = pltpu.async_copy(
                cbuf_v.at[slot],
                out_hbm.at[pl.ds((base + sp * _SUPER) * _FEAT, _SUPER * _FEAT)],
                osem,
            )
            if sp + 1 < nsuper:
                gathers = gathers_next
        for oc in out_copies:
            if oc is not None:
                oc.wait()

    return k(table, idx2d)


_PACK_CHV = 3904
_PACK_PERW = 8 * _PACK_CHV


def _sc_pack(cols, vocab):
    rem = vocab - _NUM_WORKERS * _PACK_PERW
    mesh = plsc.VectorSubcoreMesh(core_axis_name="c", subcore_axis_name="s")

    @functools.partial(
        pl.kernel,
        mesh=mesh,
        out_type=jax.ShapeDtypeStruct((vocab * _PAD_D,), jnp.float32),
        scratch_types=[
            pltpu.VMEM((2, _FEAT, _PACK_CHV), jnp.float32),
            pltpu.VMEM((2, _PACK_CHV * _PAD_D), jnp.float32),
            pltpu.SemaphoreType.DMA,
            pltpu.SemaphoreType.DMA,
        ],
        compiler_params=pltpu.CompilerParams(
            use_tc_tiling_on_sc=False, needs_layout_passes=False),
    )
    def k(c0, c1, c2, c3, c4, c5, out_hbm, in_v, pk_v, isem, osem):
        chbm = [c0, c1, c2, c3, c4, c5]
        wid = lax.axis_index("s") * _NUM_CORES + lax.axis_index("c")
        base = wid * _PACK_PERW
        lane = lax.iota(jnp.int32, _LANES)

        def fire_in(row0, nrows, slot):
            return [
                pltpu.async_copy(
                    chbm[c].at[pl.ds(row0, nrows)],
                    in_v.at[slot].at[c].at[pl.ds(0, nrows)],
                    isem,
                )
                for c in range(_FEAT)
            ]

        def pack(nrows, slot):
            pk = pk_v.at[slot]
            iv = in_v.at[slot]
            iters = nrows // _LANES

            for c in range(_FEAT):
                pos0 = lane * _PAD_D + c

                def body(i, carry, c=c, pos0=pos0):
                    v = iv[c, pl.ds(i * _LANES, _LANES)]
                    plsc.store_scatter(pk, [pos0 + i * (_LANES * _PAD_D)], v)
                    return carry

                lax.fori_loop(0, iters, body, 0)

        def chunk_rows(ch):
            if ch < 8:
                return base + ch * _PACK_CHV, _PACK_CHV
            return _NUM_WORKERS * _PACK_PERW, rem

        nch = 9 if rem else 8
        out_copies = [None, None]
        ins = fire_in(*chunk_rows(0), 0)
        for ch in range(nch):
            slot = ch % 2
            row0, nrows = chunk_rows(ch)
            if ch + 1 < nch:
                nrow0, nnrows = chunk_rows(ch + 1)
                ins_next = fire_in(nrow0, nnrows, (ch + 1) % 2)
            for i in ins:
                i.wait()
            if out_copies[slot] is not None:
                out_copies[slot].wait()
            pack(nrows, slot)
            out_copies[slot] = pltpu.async_copy(
                pk_v.at[slot].at[pl.ds(0, nrows * _PAD_D)],
                out_hbm.at[pl.ds(row0 * _PAD_D, nrows * _PAD_D)],
                osem,
            )
            if ch + 1 < nch:
                ins = ins_next
        for oc in out_copies:
            if oc is not None:
                oc.wait()

    return k(*cols)


def kernel(token_ids, frequencies, amplitudes):
    b, s = token_ids.shape
    v, nw = frequencies.shape
    n = b * s
    feat = 2 * nw
    cols = tuple(frequencies[:, i] for i in range(nw)) + tuple(
        amplitudes[:, i] for i in range(nw))
    cols = lax.optimization_barrier(cols)
    tbl_flat = _sc_pack(cols, v)
    table = tbl_flat.reshape(v, _PAD_D)
    idx_flat = token_ids.reshape(-1).astype(jnp.int32)
    idx_flat = lax.optimization_barrier(idx_flat)
    idx2d = idx_flat.reshape(n // _CHUNK, _CHUNK)
    out = _sc_gather_compact(table, idx2d, n)
    return out.reshape(b, s, feat)


# speedup vs baseline: 1.6526x; 1.2269x over previous
import functools

import jax
import jax.numpy as jnp
from jax import lax
from jax.experimental import pallas as pl
from jax.experimental.pallas import tpu as pltpu
from jax.experimental.pallas import tpu_sc as plsc

_NUM_CORES = 2
_NUM_SUBCORES = 16
_NUM_WORKERS = _NUM_CORES * _NUM_SUBCORES
_CHUNK = 800
_GRP = 2
_SUPER = _CHUNK * _GRP
_PAD_D = 8
_FEAT = 6
_LANES = 16


def _sc_gather_compact(table, idx2d, n_tokens, out_shape3):
    per_w = n_tokens // _NUM_WORKERS
    nsteps = per_w // _CHUNK
    nsuper = per_w // _SUPER
    blocks = _SUPER // _PAD_D
    mesh = plsc.VectorSubcoreMesh(core_axis_name="c", subcore_axis_name="s")

    @functools.partial(
        pl.kernel,
        mesh=mesh,
        out_type=jax.ShapeDtypeStruct(out_shape3, jnp.float32),
        scratch_types=[
            pltpu.VMEM((nsteps, _CHUNK), jnp.int32),
            pltpu.VMEM((2, _SUPER, _PAD_D), jnp.float32),
            pltpu.VMEM((2, _SUPER // 200, 200, _FEAT), jnp.float32),
            pltpu.SemaphoreType.DMA,
            pltpu.SemaphoreType.DMA,
        ],
        compiler_params=pltpu.CompilerParams(
            use_tc_tiling_on_sc=False, needs_layout_passes=False),
    )
    def k(table_hbm, idx_hbm, out3_hbm, idx_v, rows_v, cbuf_v, gsem, osem):
        rows_per_super = _SUPER // out_shape3[1]
        wid = lax.axis_index("s") * _NUM_CORES + lax.axis_index("c")
        base = wid * per_w
        b0 = wid * (per_w // out_shape3[1])
        pltpu.sync_copy(idx_hbm.at[pl.ds(wid * nsteps, nsteps)], idx_v)

        lane = lax.iota(jnp.int32, _LANES)
        def _divmod6(j):
            q = lax.shift_right_logical(j * 43691, 18)
            return q, j - q * _FEAT
        r_pat, c_pat = [], []
        for ph in range(3):
            q, r = _divmod6(lane + ph * _LANES)
            r_pat.append(q)
            c_pat.append(r)

        def fire(sp):
            slot = sp % 2
            return [
                pltpu.async_copy(
                    table_hbm.at[idx_v.at[sp * _GRP + g]],
                    rows_v.at[slot].at[pl.ds(g * _CHUNK, _CHUNK)],
                    gsem,
                )
                for g in range(_GRP)
            ]

        blocks_per_row = blocks // rows_per_super

        def compact(slot):
            rows = rows_v.at[slot]
            for r in range(rows_per_super):
                cbrow = cbuf_v.at[slot].at[r]

                def body(j, carry, r=r, cbrow=cbrow):
                    i2 = r * blocks_per_row + j
                    rbase = i2 * _PAD_D
                    for ph in range(3):
                        v = plsc.load_gather(
                            rows, [r_pat[ph] + rbase, c_pat[ph]])
                        plsc.store_scatter(
                            cbrow, [j * _PAD_D + r_pat[ph], c_pat[ph]], v)
                    return carry

                lax.fori_loop(0, blocks_per_row, body, 0)

        out_copies = [None, None]
        gathers = fire(0)
        for sp in range(nsuper):
            slot = sp % 2
            if sp + 1 < nsuper:
                gathers_next = fire(sp + 1)
            for g in gathers:
                g.wait()
            if out_copies[slot] is not None:
                out_copies[slot].wait()
            compact(slot)
            out_copies[slot] = pltpu.async_copy(
                cbuf_v.at[slot],
                out3_hbm.at[pl.ds(b0 + sp * rows_per_super, rows_per_super)],
                osem,
            )
            if sp + 1 < nsuper:
                gathers = gathers_next
        for oc in out_copies:
            if oc is not None:
                oc.wait()

    return k(table, idx2d)


_PACK_CHV = 3904
_PACK_PERW = 8 * _PACK_CHV


def _sc_pack(cols, vocab):
    rem = vocab - _NUM_WORKERS * _PACK_PERW
    mesh = plsc.VectorSubcoreMesh(core_axis_name="c", subcore_axis_name="s")

    @functools.partial(
        pl.kernel,
        mesh=mesh,
        out_type=jax.ShapeDtypeStruct((vocab * _PAD_D,), jnp.float32),
        scratch_types=[
            pltpu.VMEM((2, _FEAT, _PACK_CHV), jnp.float32),
            pltpu.VMEM((2, _PACK_CHV * _PAD_D), jnp.float32),
            pltpu.SemaphoreType.DMA,
            pltpu.SemaphoreType.DMA,
        ],
        compiler_params=pltpu.CompilerParams(
            use_tc_tiling_on_sc=False, needs_layout_passes=False),
    )
    def k(c0, c1, c2, c3, c4, c5, out_hbm, in_v, pk_v, isem, osem):
        chbm = [c0, c1, c2, c3, c4, c5]
        wid = lax.axis_index("s") * _NUM_CORES + lax.axis_index("c")
        base = wid * _PACK_PERW
        lane = lax.iota(jnp.int32, _LANES)

        def fire_in(row0, nrows, slot):
            return [
                pltpu.async_copy(
                    chbm[c].at[pl.ds(row0, nrows)],
                    in_v.at[slot].at[c].at[pl.ds(0, nrows)],
                    isem,
                )
                for c in range(_FEAT)
            ]

        def pack(nrows, slot):
            pk = pk_v.at[slot]
            iv = in_v.at[slot]
            iters = nrows // _LANES

            for c in range(_FEAT):
                pos0 = lane * _PAD_D + c

                def body(i, carry, c=c, pos0=pos0):
                    v = iv[c, pl.ds(i * _LANES, _LANES)]
                    plsc.store_scatter(pk, [pos0 + i * (_LANES * _PAD_D)], v)
                    return carry

                lax.fori_loop(0, iters, body, 0)

        def chunk_rows(ch):
            if ch < 8:
                return base + ch * _PACK_CHV, _PACK_CHV
            return _NUM_WORKERS * _PACK_PERW, rem

        nch = 9 if rem else 8
        out_copies = [None, None]
        ins = fire_in(*chunk_rows(0), 0)
        for ch in range(nch):
            slot = ch % 2
            row0, nrows = chunk_rows(ch)
            if ch + 1 < nch:
                nrow0, nnrows = chunk_rows(ch + 1)
                ins_next = fire_in(nrow0, nnrows, (ch + 1) % 2)
            for i in ins:
                i.wait()
            if out_copies[slot] is not None:
                out_copies[slot].wait()
            pack(nrows, slot)
            out_copies[slot] = pltpu.async_copy(
                pk_v.at[slot].at[pl.ds(0, nrows * _PAD_D)],
                out_hbm.at[pl.ds(row0 * _PAD_D, nrows * _PAD_D)],
                osem,
            )
            if ch + 1 < nch:
                ins = ins_next
        for oc in out_copies:
            if oc is not None:
                oc.wait()

    return k(*cols)


def kernel(token_ids, frequencies, amplitudes):
    b, s = token_ids.shape
    v, nw = frequencies.shape
    n = b * s
    feat = 2 * nw
    cols = tuple(frequencies[:, i] for i in range(nw)) + tuple(
        amplitudes[:, i] for i in range(nw))
    cols = lax.optimization_barrier(cols)
    tbl_flat = _sc_pack(cols, v)
    table = tbl_flat.reshape(v, _PAD_D)
    idx_flat = token_ids.reshape(-1).astype(jnp.int32)
    idx_flat = lax.optimization_barrier(idx_flat)
    idx2d = idx_flat.reshape(n // _CHUNK, _CHUNK)
    return _sc_gather_compact(table, idx2d, n, (b, s, feat))


# speedup vs baseline: 4.3336x; 2.6223x over previous
import functools

import jax
import jax.numpy as jnp
from jax import lax
from jax.experimental import pallas as pl
from jax.experimental.pallas import tpu as pltpu
from jax.experimental.pallas import tpu_sc as plsc

_NUM_CORES = 2
_NUM_SUBCORES = 16
_NUM_WORKERS = _NUM_CORES * _NUM_SUBCORES
_CHUNK = 800
_GRP = 2
_SUPER = _CHUNK * _GRP
_PAD_D = 8
_FEAT = 6
_LANES = 16


def _sc_gather_planes(table, idx3, out_shape3):
    feat, seq, batch = out_shape3
    n_sg, n_bh = 8, 4
    s_per = seq // n_sg
    b_per = batch // n_bh
    nsuper = 8
    b_sub = b_per // nsuper
    toks = s_per * b_sub
    mesh = plsc.VectorSubcoreMesh(core_axis_name="c", subcore_axis_name="s")

    @functools.partial(
        pl.kernel,
        mesh=mesh,
        out_type=jax.ShapeDtypeStruct(out_shape3, jnp.float32),
        scratch_types=[
            pltpu.VMEM((nsuper, toks), jnp.int32),
            pltpu.VMEM((2, toks, _PAD_D), jnp.float32),
            pltpu.VMEM((2, feat, s_per, b_sub), jnp.float32),
            pltpu.SemaphoreType.DMA,
            pltpu.SemaphoreType.DMA,
        ],
        compiler_params=pltpu.CompilerParams(
            use_tc_tiling_on_sc=False, needs_layout_passes=False),
    )
    def k(table_hbm, idx_hbm, out_hbm, idx_v, rows_v, cbuf_v, gsem, osem):
        wid = lax.axis_index("s") * _NUM_CORES + lax.axis_index("c")
        sg = wid // n_bh
        bh = wid - sg * n_bh
        lane = lax.iota(jnp.int32, _LANES)
        pltpu.sync_copy(idx_hbm.at[wid], idx_v)

        def fire(sb):
            slot = sb % 2
            half = toks // _GRP
            return [
                pltpu.async_copy(
                    table_hbm.at[idx_v.at[sb].at[pl.ds(g * half, half)]],
                    rows_v.at[slot].at[pl.ds(g * half, half)],
                    gsem,
                )
                for g in range(_GRP)
            ]

        def compact(slot):
            rows = rows_v.at[slot]
            for c in range(feat):
                cb_c = cbuf_v.at[slot].at[c]
                c_vec = lane * 0 + c

                def body(s_in, carry, cb_c=cb_c, c_vec=c_vec):
                    s_vec = lane * 0 + s_in
                    for b0 in range(0, b_sub, _LANES):
                        v = plsc.load_gather(
                            rows, [s_in * b_sub + b0 + lane, c_vec])
                        plsc.store_scatter(cb_c, [s_vec, b0 + lane], v)
                    return carry

                lax.fori_loop(0, s_per, body, 0)

        out_copies = [None, None]
        gathers = fire(0)
        for sb in range(nsuper):
            slot = sb % 2
            if sb + 1 < nsuper:
                gathers_next = fire(sb + 1)
            for g in gathers:
                g.wait()
            if out_copies[slot] is not None:
                out_copies[slot].wait()
            compact(slot)
            out_copies[slot] = pltpu.async_copy(
                cbuf_v.at[slot],
                out_hbm.at[:, pl.ds(sg * s_per, s_per),
                           pl.ds(bh * b_per + sb * b_sub, b_sub)],
                osem,
            )
            if sb + 1 < nsuper:
                gathers = gathers_next
        for oc in out_copies:
            if oc is not None:
                oc.wait()

    return k(table, idx3)


_PACK_CHV = 3904
_PACK_PERW = 8 * _PACK_CHV


def _sc_pack(cols, vocab):
    rem = vocab - _NUM_WORKERS * _PACK_PERW
    mesh = plsc.VectorSubcoreMesh(core_axis_name="c", subcore_axis_name="s")

    @functools.partial(
        pl.kernel,
        mesh=mesh,
        out_type=jax.ShapeDtypeStruct((vocab * _PAD_D,), jnp.float32),
        scratch_types=[
            pltpu.VMEM((2, _FEAT, _PACK_CHV), jnp.float32),
            pltpu.VMEM((2, _PACK_CHV * _PAD_D), jnp.float32),
            pltpu.SemaphoreType.DMA,
            pltpu.SemaphoreType.DMA,
        ],
        compiler_params=pltpu.CompilerParams(
            use_tc_tiling_on_sc=False, needs_layout_passes=False),
    )
    def k(c0, c1, c2, c3, c4, c5, out_hbm, in_v, pk_v, isem, osem):
        chbm = [c0, c1, c2, c3, c4, c5]
        wid = lax.axis_index("s") * _NUM_CORES + lax.axis_index("c")
        base = wid * _PACK_PERW
        lane = lax.iota(jnp.int32, _LANES)

        def fire_in(row0, nrows, slot):
            return [
                pltpu.async_copy(
                    chbm[c].at[pl.ds(row0, nrows)],
                    in_v.at[slot].at[c].at[pl.ds(0, nrows)],
                    isem,
                )
                for c in range(_FEAT)
            ]

        def pack(nrows, slot):
            pk = pk_v.at[slot]
            iv = in_v.at[slot]
            iters = nrows // _LANES

            for c in range(_FEAT):
                pos0 = lane * _PAD_D + c

                def body(i, carry, c=c, pos0=pos0):
                    v = iv[c, pl.ds(i * _LANES, _LANES)]
                    plsc.store_scatter(pk, [pos0 + i * (_LANES * _PAD_D)], v)
                    return carry

                lax.fori_loop(0, iters, body, 0)

        def chunk_rows(ch):
            if ch < 8:
                return base + ch * _PACK_CHV, _PACK_CHV
            return _NUM_WORKERS * _PACK_PERW, rem

        nch = 9 if rem else 8
        out_copies = [None, None]
        ins = fire_in(*chunk_rows(0), 0)
        for ch in range(nch):
            slot = ch % 2
            row0, nrows = chunk_rows(ch)
            if ch + 1 < nch:
                nrow0, nnrows = chunk_rows(ch + 1)
                ins_next = fire_in(nrow0, nnrows, (ch + 1) % 2)
            for i in ins:
                i.wait()
            if out_copies[slot] is not None:
                out_copies[slot].wait()
            pack(nrows, slot)
            out_copies[slot] = pltpu.async_copy(
                pk_v.at[slot].at[pl.ds(0, nrows * _PAD_D)],
                out_hbm.at[pl.ds(row0 * _PAD_D, nrows * _PAD_D)],
                osem,
            )
            if ch + 1 < nch:
                ins = ins_next
        for oc in out_copies:
            if oc is not None:
                oc.wait()

    return k(*cols)


def kernel(token_ids, frequencies, amplitudes):
    b, s = token_ids.shape
    v, nw = frequencies.shape
    feat = 2 * nw
    cols = tuple(frequencies[:, i] for i in range(nw)) + tuple(
        amplitudes[:, i] for i in range(nw))
    cols = lax.optimization_barrier(cols)
    tbl_flat = _sc_pack(cols, v)
    table = tbl_flat.reshape(v, _PAD_D)
    idx_flat = (token_ids.T.astype(jnp.int32)
                .reshape(8, s // 8, 4, 8, b // 32)
                .transpose(0, 2, 3, 1, 4)
                .reshape(-1))
    idx_flat = lax.optimization_barrier(idx_flat)
    idx3 = idx_flat.reshape(32, 8, (s // 8) * (b // 32))
    out_t = _sc_gather_planes(table, idx3, (feat, s, b))
    return jnp.transpose(out_t, (2, 1, 0))


# speedup vs baseline: 5.0368x; 1.1622x over previous
import functools

import jax
import jax.numpy as jnp
from jax import lax
from jax.experimental import pallas as pl
from jax.experimental.pallas import tpu as pltpu
from jax.experimental.pallas import tpu_sc as plsc

_NUM_CORES = 2
_NUM_SUBCORES = 16
_NUM_WORKERS = _NUM_CORES * _NUM_SUBCORES
_CHUNK = 800
_GRP = 2
_SUPER = _CHUNK * _GRP
_PAD_D = 8
_FEAT = 6
_LANES = 16


def _sc_gather_planes(table, idx3, out_shape3):
    feat, seq, batch = out_shape3
    n_sg, n_bh = 8, 4
    s_per = seq // n_sg
    b_per = batch // n_bh
    nsuper = 8
    b_sub = b_per // nsuper
    toks = s_per * b_sub
    mesh = plsc.VectorSubcoreMesh(core_axis_name="c", subcore_axis_name="s")

    @functools.partial(
        pl.kernel,
        mesh=mesh,
        out_type=jax.ShapeDtypeStruct(out_shape3, jnp.float32),
        scratch_types=[
            pltpu.VMEM((nsuper, toks), jnp.int32),
            pltpu.VMEM((2, toks, _PAD_D), jnp.float32),
            pltpu.VMEM((2, feat, s_per, b_sub), jnp.float32),
            pltpu.SemaphoreType.DMA,
            pltpu.SemaphoreType.DMA,
        ],
        compiler_params=pltpu.CompilerParams(
            use_tc_tiling_on_sc=False, needs_layout_passes=False),
    )
    def k(table_hbm, idx_hbm, out_hbm, idx_v, rows_v, cbuf_v, gsem, osem):
        wid = lax.axis_index("s") * _NUM_CORES + lax.axis_index("c")
        sg = wid // n_bh
        bh = wid - sg * n_bh
        lane = lax.iota(jnp.int32, _LANES)
        pltpu.sync_copy(idx_hbm.at[wid], idx_v)

        def fire(sb):
            slot = sb % 2
            half = toks // _GRP
            return [
                pltpu.async_copy(
                    table_hbm.at[idx_v.at[sb].at[pl.ds(g * half, half)]],
                    rows_v.at[slot].at[pl.ds(g * half, half)],
                    gsem,
                )
                for g in range(_GRP)
            ]

        def compact(slot):
            rows = rows_v.at[slot]
            for c in range(feat):
                cb_c = cbuf_v.at[slot].at[c]
                c_vec = lane * 0 + c

                def body(s_in, carry, cb_c=cb_c, c_vec=c_vec):
                    s_vec = lane * 0 + s_in
                    for b0 in range(0, b_sub, _LANES):
                        v = plsc.load_gather(
                            rows, [s_in * b_sub + b0 + lane, c_vec])
                        plsc.store_scatter(cb_c, [s_vec, b0 + lane], v)
                    return carry

                lax.fori_loop(0, s_per, body, 0)

        out_copies = [None, None]
        gathers = fire(0)
        for sb in range(nsuper):
            slot = sb % 2
            if sb + 1 < nsuper:
                gathers_next = fire(sb + 1)
            for g in gathers:
                g.wait()
            if out_copies[slot] is not None:
                out_copies[slot].wait()
            compact(slot)
            out_copies[slot] = pltpu.async_copy(
                cbuf_v.at[slot],
                out_hbm.at[:, pl.ds(sg * s_per, s_per),
                           pl.ds(bh * b_per + sb * b_sub, b_sub)],
                osem,
            )
            if sb + 1 < nsuper:
                gathers = gathers_next
        for oc in out_copies:
            if oc is not None:
                oc.wait()

    return k(table, idx3)


_PACK_CHV = 3904
_PACK_PERW = 8 * _PACK_CHV


def _sc_pack(cols, vocab):
    rem = vocab - _NUM_WORKERS * _PACK_PERW
    mesh = plsc.VectorSubcoreMesh(core_axis_name="c", subcore_axis_name="s")

    @functools.partial(
        pl.kernel,
        mesh=mesh,
        out_type=jax.ShapeDtypeStruct((vocab * _PAD_D,), jnp.float32),
        scratch_types=[
            pltpu.VMEM((2, _FEAT, _PACK_CHV), jnp.float32),
            pltpu.VMEM((2, _PACK_CHV * _PAD_D), jnp.float32),
            pltpu.SemaphoreType.DMA,
            pltpu.SemaphoreType.DMA,
        ],
        compiler_params=pltpu.CompilerParams(
            use_tc_tiling_on_sc=False, needs_layout_passes=False),
    )
    def k(f3, a3, out_hbm, in_v, pk_v, isem, osem):
        wid = lax.axis_index("s") * _NUM_CORES + lax.axis_index("c")
        base = wid * _PACK_PERW
        lane = lax.iota(jnp.int32, _LANES)

        def fire_in(row0, nrows, slot):
            half = _FEAT // 2
            return [
                pltpu.async_copy(
                    (f3 if c < half else a3).at[
                        pl.ds((c % half) * vocab + row0, nrows)],
                    in_v.at[slot].at[c].at[pl.ds(0, nrows)],
                    isem,
                )
                for c in range(_FEAT)
            ]

        def pack(nrows, slot):
            pk = pk_v.at[slot]
            iv = in_v.at[slot]
            iters = nrows // _LANES

            for c in range(_FEAT):
                pos0 = lane * _PAD_D + c

                def body(i, carry, c=c, pos0=pos0):
                    v = iv[c, pl.ds(i * _LANES, _LANES)]
                    plsc.store_scatter(pk, [pos0 + i * (_LANES * _PAD_D)], v)
                    return carry

                lax.fori_loop(0, iters, body, 0)

        def chunk_rows(ch):
            if ch < 8:
                return base + ch * _PACK_CHV, _PACK_CHV
            return _NUM_WORKERS * _PACK_PERW, rem

        nch = 9 if rem else 8
        out_copies = [None, None]
        ins = fire_in(*chunk_rows(0), 0)
        for ch in range(nch):
            slot = ch % 2
            row0, nrows = chunk_rows(ch)
            if ch + 1 < nch:
                nrow0, nnrows = chunk_rows(ch + 1)
                ins_next = fire_in(nrow0, nnrows, (ch + 1) % 2)
            for i in ins:
                i.wait()
            if out_copies[slot] is not None:
                out_copies[slot].wait()
            pack(nrows, slot)
            out_copies[slot] = pltpu.async_copy(
                pk_v.at[slot].at[pl.ds(0, nrows * _PAD_D)],
                out_hbm.at[pl.ds(row0 * _PAD_D, nrows * _PAD_D)],
                osem,
            )
            if ch + 1 < nch:
                ins = ins_next
        for oc in out_copies:
            if oc is not None:
                oc.wait()

    return k(*cols)


def kernel(token_ids, frequencies, amplitudes):
    b, s = token_ids.shape
    v, nw = frequencies.shape
    feat = 2 * nw
    cols = (frequencies.T.reshape(-1), amplitudes.T.reshape(-1))
    cols = lax.optimization_barrier(cols)
    tbl_flat = _sc_pack(cols, v)
    table = tbl_flat.reshape(v, _PAD_D)
    idx_flat = (token_ids.T.astype(jnp.int32)
                .reshape(8, s // 8, 4, 8, b // 32)
                .transpose(0, 2, 3, 1, 4)
                .reshape(-1))
    idx_flat = lax.optimization_barrier(idx_flat)
    idx3 = idx_flat.reshape(32, 8, (s // 8) * (b // 32))
    out_t = _sc_gather_planes(table, idx3, (feat, s, b))
    return jnp.transpose(out_t, (2, 1, 0))
